# baseline (speedup 1.0000x reference)
import functools

import jax
import jax.numpy as jnp
from jax import lax
from jax.experimental import pallas as pl
from jax.experimental.pallas import tpu as pltpu
from jax.experimental.pallas import tpu_sc as plsc

T = 2048
D = 768
F = 2048
E = 8
K = 2
BM = 512
A = T * K
A_PAD = A + E * BM
NB = A_PAD // BM


def _router_body(x_ref, rw_ref, pos_ref, wts_ref, be_ref, bv_ref):
    x = x_ref[...]
    logits = jnp.dot(x, rw_ref[...], preferred_element_type=jnp.float32)
    m = jnp.max(logits, axis=1, keepdims=True)
    ex = jnp.exp(logits - m)
    probs = ex / jnp.sum(ex, axis=1, keepdims=True)

    iota_e = jax.lax.broadcasted_iota(jnp.int32, (T, E), 1)
    m1 = jnp.max(probs, axis=1, keepdims=True)
    i1 = jnp.min(jnp.where(probs == m1, iota_e, E), axis=1, keepdims=True)
    pm = jnp.where(iota_e == i1, -1.0, probs)
    m2 = jnp.max(pm, axis=1, keepdims=True)
    i2 = jnp.min(jnp.where(pm == m2, iota_e, E), axis=1, keepdims=True)
    sw = m1 + m2
    w1 = m1 / sw
    w2 = m2 / sw
    wts_ref[...] = jnp.concatenate([w1, w2], axis=1)

    h1 = (iota_e == i1).astype(jnp.float32)
    h2 = (iota_e == i2).astype(jnp.float32)
    cc = jnp.concatenate([h1, h2], axis=1)
    sh = 1
    while sh < T:
        cc = cc + jnp.concatenate(
            [jnp.zeros((sh, 2 * E), jnp.float32), cc[:T - sh]], axis=0)
        sh *= 2
    c1 = cc[:, :E]
    c2 = cc[:, E:]
    tot1 = c1[T - 1:T, :]
    tot2 = c2[T - 1:T, :]
    counts = tot1 + tot2
    nblk = jnp.ceil(counts / BM)
    tri_ex = (jax.lax.broadcasted_iota(jnp.int32, (E, E), 0)
              < jax.lax.broadcasted_iota(jnp.int32, (E, E), 1)).astype(jnp.float32)
    tri_in = (jax.lax.broadcasted_iota(jnp.int32, (E, E), 0)
              <= jax.lax.broadcasted_iota(jnp.int32, (E, E), 1)).astype(jnp.float32)
    starts = jnp.dot(nblk * BM, tri_ex, preferred_element_type=jnp.float32)
    end_blk = jnp.dot(nblk, tri_in, preferred_element_type=jnp.float32)

    rank1 = jnp.sum(jnp.where(iota_e == i1, c1 - 1.0, 0.0), axis=1, keepdims=True)
    rank2 = jnp.sum(jnp.where(iota_e == i2, tot1 + c2 - 1.0, 0.0), axis=1,
                    keepdims=True)
    s1 = jnp.sum(jnp.where(iota_e == i1, starts, 0.0), axis=1, keepdims=True)
    s2 = jnp.sum(jnp.where(iota_e == i2, starts, 0.0), axis=1, keepdims=True)
    pos1 = (s1 + rank1).astype(jnp.int32)
    pos2 = (s2 + rank2).astype(jnp.int32)
    pos_ref[...] = jnp.concatenate([pos1, pos2], axis=1)

    end_blk_i = end_blk.astype(jnp.int32)
    bi = jax.lax.broadcasted_iota(jnp.int32, (NB, E), 0)
    ge = (bi >= end_blk_i).astype(jnp.int32)
    bexp = jnp.minimum(jnp.sum(ge, axis=1, keepdims=True), E - 1)
    total_used = jnp.sum(end_blk_i[:1, E - 1:E])
    bvalid = (jax.lax.broadcasted_iota(jnp.int32, (NB, 1), 0)
              < total_used).astype(jnp.int32)
    be_ref[...] = bexp
    bv_ref[...] = bvalid


_SC_MESH = plsc.VectorSubcoreMesh(core_axis_name="c", subcore_axis_name="s")
_NW = 32
_TPW = T // _NW
_APW = A // _NW


@functools.partial(
    pl.kernel,
    out_type=jax.ShapeDtypeStruct((A_PAD, D), jnp.float32),
    mesh=_SC_MESH,
    scratch_types=[
        pltpu.VMEM((_TPW,), jnp.int32),
        pltpu.VMEM((_TPW,), jnp.int32),
        pltpu.VMEM((_TPW, D), jnp.float32),
        pltpu.SemaphoreType.DMA,
    ],
)
def _sc_scatter(x_hbm, idx_hbm, out_hbm, idx1_v, idx2_v, rows_v, sem):
    wid = lax.axis_index("s") * 2 + lax.axis_index("c")
    pltpu.sync_copy(idx_hbm.at[wid], idx1_v)
    pltpu.sync_copy(idx_hbm.at[wid + _NW], idx2_v)
    pltpu.sync_copy(x_hbm.at[pl.ds(wid * _TPW, _TPW)], rows_v)
    c1 = pltpu.async_copy(rows_v, out_hbm.at[idx1_v], sem)
    c2 = pltpu.async_copy(rows_v, out_hbm.at[idx2_v], sem)
    c1.wait()
    c2.wait()


@functools.partial(
    pl.kernel,
    out_type=jax.ShapeDtypeStruct((A, D), jnp.float32),
    mesh=_SC_MESH,
    scratch_types=[
        pltpu.VMEM((_APW,), jnp.int32),
        pltpu.VMEM((_APW, D), jnp.float32),
        pltpu.SemaphoreType.DMA,
    ],
)
def _sc_gather(y_hbm, idx_hbm, out_hbm, idx_v, rows_v, sem):
    wid = lax.axis_index("s") * 2 + lax.axis_index("c")
    pltpu.sync_copy(idx_hbm.at[wid], idx_v)
    pltpu.async_copy(y_hbm.at[idx_v], rows_v, sem).wait()
    pltpu.sync_copy(rows_v, out_hbm.at[pl.ds(wid * _APW, _APW)])


def _gmm_body(be_ref, bv_ref, x_ref, wg_ref, wu_ref, wd_ref, y_ref):
    i = pl.program_id(0)

    @pl.when(bv_ref[i] == 1)
    def _():
        xb = x_ref[...]
        g = jnp.dot(xb, wg_ref[0], preferred_element_type=jnp.float32)
        u = jnp.dot(xb, wu_ref[0], preferred_element_type=jnp.float32)
        h = g * jax.nn.sigmoid(g) * u
        y_ref[...] = jnp.dot(h, wd_ref[0], preferred_element_type=jnp.float32)


def _combine_body(g1_ref, g2_ref, wts_ref, o_ref):
    w = wts_ref[...]
    o_ref[...] = g1_ref[...] * w[:, 0:1] + g2_ref[...] * w[:, 1:2]


def kernel(hidden_states, router_w, w_gate, w_up, w_down):
    b, s, d = hidden_states.shape
    x = hidden_states.reshape(T, D)

    pos, wts, bexp, bvalid = pl.pallas_call(
        _router_body,
        out_shape=[
            jax.ShapeDtypeStruct((T, K), jnp.int32),
            jax.ShapeDtypeStruct((T, K), jnp.float32),
            jax.ShapeDtypeStruct((NB, 1), jnp.int32),
            jax.ShapeDtypeStruct((NB, 1), jnp.int32),
        ],
    )(x, router_w)

    idx_scatter = pos.T.reshape(2 * _NW, _TPW)
    idx_gather = pos.T.reshape(_NW, _APW)
    bexp = bexp.reshape(NB)
    bvalid = bvalid.reshape(NB)

    x_sorted = _sc_scatter(x, idx_scatter)

    y_sorted = pl.pallas_call(
        _gmm_body,
        grid_spec=pltpu.PrefetchScalarGridSpec(
            num_scalar_prefetch=2,
            grid=(NB,),
            in_specs=[
                pl.BlockSpec((BM, D), lambda i, be, bv: (i, 0)),
                pl.BlockSpec((1, D, F), lambda i, be, bv: (be[i], 0, 0)),
                pl.BlockSpec((1, D, F), lambda i, be, bv: (be[i], 0, 0)),
                pl.BlockSpec((1, F, D), lambda i, be, bv: (be[i], 0, 0)),
            ],
            out_specs=pl.BlockSpec((BM, D), lambda i, be, bv: (i, 0)),
        ),
        out_shape=jax.ShapeDtypeStruct((A_PAD, D), jnp.float32),
    )(bexp, bvalid, x_sorted, w_gate, w_up, w_down)

    g_all = _sc_gather(y_sorted, idx_gather)

    BC = 256
    out = pl.pallas_call(
        _combine_body,
        grid=(T // BC,),
        in_specs=[
            pl.BlockSpec((BC, D), lambda i: (i, 0)),
            pl.BlockSpec((BC, D), lambda i: (i + T // BC, 0)),
            pl.BlockSpec((BC, K), lambda i: (i, 0)),
        ],
        out_specs=pl.BlockSpec((BC, D), lambda i: (i, 0)),
        out_shape=jax.ShapeDtypeStruct((T, D), jnp.float32),
    )(g_all, g_all, wts)

    return out.reshape(b, s, d)


# speedup vs baseline: 1.0135x; 1.0135x over previous
import functools

import jax
import jax.numpy as jnp
from jax import lax
from jax.experimental import pallas as pl
from jax.experimental.pallas import tpu as pltpu
from jax.experimental.pallas import tpu_sc as plsc

T = 2048
D = 768
F = 2048
E = 8
K = 2
BM = 512
A = T * K
A_PAD = A + E * BM
NB = A_PAD // BM


def _router_body(x_ref, rw_ref, pos_ref, wts_ref, be_ref, bv_ref):
    x = x_ref[...]
    logits = jnp.dot(x, rw_ref[...], preferred_element_type=jnp.float32)
    m = jnp.max(logits, axis=1, keepdims=True)
    ex = jnp.exp(logits - m)
    probs = ex / jnp.sum(ex, axis=1, keepdims=True)

    iota_e = jax.lax.broadcasted_iota(jnp.int32, (T, E), 1)
    m1 = jnp.max(probs, axis=1, keepdims=True)
    i1 = jnp.min(jnp.where(probs == m1, iota_e, E), axis=1, keepdims=True)
    pm = jnp.where(iota_e == i1, -1.0, probs)
    m2 = jnp.max(pm, axis=1, keepdims=True)
    i2 = jnp.min(jnp.where(pm == m2, iota_e, E), axis=1, keepdims=True)
    sw = m1 + m2
    w1 = m1 / sw
    w2 = m2 / sw
    wts_ref[...] = jnp.concatenate([w1, w2], axis=1)

    h1 = (iota_e == i1).astype(jnp.float32)
    h2 = (iota_e == i2).astype(jnp.float32)
    cc = jnp.concatenate([h1, h2], axis=1)
    sh = 1
    while sh < T:
        cc = cc + jnp.concatenate(
            [jnp.zeros((sh, 2 * E), jnp.float32), cc[:T - sh]], axis=0)
        sh *= 2
    c1 = cc[:, :E]
    c2 = cc[:, E:]
    tot1 = c1[T - 1:T, :]
    tot2 = c2[T - 1:T, :]
    counts = tot1 + tot2
    nblk = jnp.ceil(counts / BM)
    tri_ex = (jax.lax.broadcasted_iota(jnp.int32, (E, E), 0)
              < jax.lax.broadcasted_iota(jnp.int32, (E, E), 1)).astype(jnp.float32)
    tri_in = (jax.lax.broadcasted_iota(jnp.int32, (E, E), 0)
              <= jax.lax.broadcasted_iota(jnp.int32, (E, E), 1)).astype(jnp.float32)
    starts = jnp.dot(nblk * BM, tri_ex, preferred_element_type=jnp.float32)
    end_blk = jnp.dot(nblk, tri_in, preferred_element_type=jnp.float32)

    rank1 = jnp.sum(jnp.where(iota_e == i1, c1 - 1.0, 0.0), axis=1, keepdims=True)
    rank2 = jnp.sum(jnp.where(iota_e == i2, tot1 + c2 - 1.0, 0.0), axis=1,
                    keepdims=True)
    s1 = jnp.sum(jnp.where(iota_e == i1, starts, 0.0), axis=1, keepdims=True)
    s2 = jnp.sum(jnp.where(iota_e == i2, starts, 0.0), axis=1, keepdims=True)
    pos1 = (s1 + rank1).astype(jnp.int32)
    pos2 = (s2 + rank2).astype(jnp.int32)
    pos_ref[...] = jnp.concatenate([pos1, pos2], axis=1)

    end_blk_i = end_blk.astype(jnp.int32)
    bi = jax.lax.broadcasted_iota(jnp.int32, (NB, E), 0)
    ge = (bi >= end_blk_i).astype(jnp.int32)
    bexp = jnp.minimum(jnp.sum(ge, axis=1, keepdims=True), E - 1)
    total_used = jnp.sum(end_blk_i[:1, E - 1:E])
    bvalid = (jax.lax.broadcasted_iota(jnp.int32, (NB, 1), 0)
              < total_used).astype(jnp.int32)
    be_ref[...] = bexp
    bv_ref[...] = bvalid


_SC_MESH = plsc.VectorSubcoreMesh(core_axis_name="c", subcore_axis_name="s")
_NW = 32
_TPW = T // _NW
_APW = A // _NW


@functools.partial(
    pl.kernel,
    out_type=jax.ShapeDtypeStruct((A_PAD, D), jnp.float32),
    mesh=_SC_MESH,
    scratch_types=[
        pltpu.VMEM((_TPW,), jnp.int32),
        pltpu.VMEM((_TPW,), jnp.int32),
        pltpu.VMEM((_TPW, D), jnp.float32),
        pltpu.SemaphoreType.DMA,
    ],
)
def _sc_scatter(x_hbm, idx_hbm, out_hbm, idx1_v, idx2_v, rows_v, sem):
    wid = lax.axis_index("s") * 2 + lax.axis_index("c")
    pltpu.sync_copy(idx_hbm.at[wid], idx1_v)
    pltpu.sync_copy(idx_hbm.at[wid + _NW], idx2_v)
    pltpu.sync_copy(x_hbm.at[pl.ds(wid * _TPW, _TPW)], rows_v)
    c1 = pltpu.async_copy(rows_v, out_hbm.at[idx1_v], sem)
    c2 = pltpu.async_copy(rows_v, out_hbm.at[idx2_v], sem)
    c1.wait()
    c2.wait()


@functools.partial(
    pl.kernel,
    out_type=jax.ShapeDtypeStruct((A, D), jnp.float32),
    mesh=_SC_MESH,
    scratch_types=[
        pltpu.VMEM((_APW,), jnp.int32),
        pltpu.VMEM((_APW, D), jnp.float32),
        pltpu.SemaphoreType.DMA,
    ],
)
def _sc_gather(y_hbm, idx_hbm, out_hbm, idx_v, rows_v, sem):
    wid = lax.axis_index("s") * 2 + lax.axis_index("c")
    pltpu.sync_copy(idx_hbm.at[wid], idx_v)
    pltpu.async_copy(y_hbm.at[idx_v], rows_v, sem).wait()
    pltpu.sync_copy(rows_v, out_hbm.at[pl.ds(wid * _APW, _APW)])


def _gmm_body(be_ref, bv_ref, x_ref, wg_ref, wu_ref, wd_ref, y_ref):
    i = pl.program_id(0)

    @pl.when(bv_ref[i] == 1)
    def _():
        xb = x_ref[...].astype(jnp.bfloat16)
        wg = wg_ref[0].astype(jnp.bfloat16)
        wu = wu_ref[0].astype(jnp.bfloat16)
        wd = wd_ref[0].astype(jnp.bfloat16)
        g = jnp.dot(xb, wg, preferred_element_type=jnp.float32)
        u = jnp.dot(xb, wu, preferred_element_type=jnp.float32)
        h = (g * jax.nn.sigmoid(g) * u).astype(jnp.bfloat16)
        y_ref[...] = jnp.dot(h, wd, preferred_element_type=jnp.float32)


def _combine_body(g1_ref, g2_ref, wts_ref, o_ref):
    w = wts_ref[...]
    o_ref[...] = g1_ref[...] * w[:, 0:1] + g2_ref[...] * w[:, 1:2]


def kernel(hidden_states, router_w, w_gate, w_up, w_down):
    b, s, d = hidden_states.shape
    x = hidden_states.reshape(T, D)

    pos, wts, bexp, bvalid = pl.pallas_call(
        _router_body,
        out_shape=[
            jax.ShapeDtypeStruct((T, K), jnp.int32),
            jax.ShapeDtypeStruct((T, K), jnp.float32),
            jax.ShapeDtypeStruct((NB, 1), jnp.int32),
            jax.ShapeDtypeStruct((NB, 1), jnp.int32),
        ],
    )(x, router_w)

    idx_scatter = pos.T.reshape(2 * _NW, _TPW)
    idx_gather = pos.T.reshape(_NW, _APW)
    bexp = bexp.reshape(NB)
    bvalid = bvalid.reshape(NB)

    x_sorted = _sc_scatter(x, idx_scatter)

    y_sorted = pl.pallas_call(
        _gmm_body,
        grid_spec=pltpu.PrefetchScalarGridSpec(
            num_scalar_prefetch=2,
            grid=(NB,),
            in_specs=[
                pl.BlockSpec((BM, D), lambda i, be, bv: (i, 0)),
                pl.BlockSpec((1, D, F), lambda i, be, bv: (be[i], 0, 0)),
                pl.BlockSpec((1, D, F), lambda i, be, bv: (be[i], 0, 0)),
                pl.BlockSpec((1, F, D), lambda i, be, bv: (be[i], 0, 0)),
            ],
            out_specs=pl.BlockSpec((BM, D), lambda i, be, bv: (i, 0)),
        ),
        out_shape=jax.ShapeDtypeStruct((A_PAD, D), jnp.float32),
    )(bexp, bvalid, x_sorted, w_gate, w_up, w_down)

    g_all = _sc_gather(y_sorted, idx_gather)

    BC = 512
    out = pl.pallas_call(
        _combine_body,
        grid=(T // BC,),
        in_specs=[
            pl.BlockSpec((BC, D), lambda i: (i, 0)),
            pl.BlockSpec((BC, D), lambda i: (i + T // BC, 0)),
            pl.BlockSpec((BC, K), lambda i: (i, 0)),
        ],
        out_specs=pl.BlockSpec((BC, D), lambda i: (i, 0)),
        out_shape=jax.ShapeDtypeStruct((T, D), jnp.float32),
    )(g_all, g_all, wts)

    return out.reshape(b, s, d)


# speedup vs baseline: 1.0198x; 1.0062x over previous
import functools

import jax
import jax.numpy as jnp
from jax import lax
from jax.experimental import pallas as pl
from jax.experimental.pallas import tpu as pltpu
from jax.experimental.pallas import tpu_sc as plsc

T = 2048
D = 768
F = 2048
E = 8
K = 2
BM = 512
A = T * K
A_PAD = A + E * BM
NB = A_PAD // BM


def _router_body(x_ref, rw_ref, pos_ref, wts_ref, be_ref, bv_ref):
    x = x_ref[0]
    logits = jnp.dot(x, rw_ref[...], preferred_element_type=jnp.float32)
    m = jnp.max(logits, axis=1, keepdims=True)
    ex = jnp.exp(logits - m)
    probs = ex / jnp.sum(ex, axis=1, keepdims=True)

    iota_e = jax.lax.broadcasted_iota(jnp.int32, (T, E), 1)
    m1 = jnp.max(probs, axis=1, keepdims=True)
    i1 = jnp.min(jnp.where(probs == m1, iota_e, E), axis=1, keepdims=True)
    pm = jnp.where(iota_e == i1, -1.0, probs)
    m2 = jnp.max(pm, axis=1, keepdims=True)
    i2 = jnp.min(jnp.where(pm == m2, iota_e, E), axis=1, keepdims=True)
    sw = m1 + m2
    w1 = m1 / sw
    w2 = m2 / sw
    wts_ref[...] = jnp.concatenate([w1, w2], axis=1)

    h1 = (iota_e == i1).astype(jnp.float32)
    h2 = (iota_e == i2).astype(jnp.float32)
    cc = jnp.concatenate([h1, h2], axis=1)
    sh = 1
    while sh < T:
        cc = cc + jnp.concatenate(
            [jnp.zeros((sh, 2 * E), jnp.float32), cc[:T - sh]], axis=0)
        sh *= 2
    c1 = cc[:, :E]
    c2 = cc[:, E:]
    tot1 = c1[T - 1:T, :]
    tot2 = c2[T - 1:T, :]
    counts = tot1 + tot2
    nblk = jnp.ceil(counts / BM)
    tri_ex = (jax.lax.broadcasted_iota(jnp.int32, (E, E), 0)
              < jax.lax.broadcasted_iota(jnp.int32, (E, E), 1)).astype(jnp.float32)
    tri_in = (jax.lax.broadcasted_iota(jnp.int32, (E, E), 0)
              <= jax.lax.broadcasted_iota(jnp.int32, (E, E), 1)).astype(jnp.float32)
    starts = jnp.dot(nblk * BM, tri_ex, preferred_element_type=jnp.float32)
    end_blk = jnp.dot(nblk, tri_in, preferred_element_type=jnp.float32)

    rank1 = jnp.sum(jnp.where(iota_e == i1, c1 - 1.0, 0.0), axis=1, keepdims=True)
    rank2 = jnp.sum(jnp.where(iota_e == i2, tot1 + c2 - 1.0, 0.0), axis=1,
                    keepdims=True)
    s1 = jnp.sum(jnp.where(iota_e == i1, starts, 0.0), axis=1, keepdims=True)
    s2 = jnp.sum(jnp.where(iota_e == i2, starts, 0.0), axis=1, keepdims=True)
    pos1 = (s1 + rank1).astype(jnp.int32)
    pos2 = (s2 + rank2).astype(jnp.int32)
    pos_ref[...] = jnp.concatenate(
        [pos1.reshape(1, T), pos2.reshape(1, T)], axis=0)

    end_blk_i = end_blk.astype(jnp.int32)
    bi = jax.lax.broadcasted_iota(jnp.int32, (NB, E), 0)
    ge = (bi >= end_blk_i).astype(jnp.int32)
    bexp = jnp.minimum(jnp.sum(ge, axis=1, keepdims=True), E - 1)
    total_used = jnp.sum(end_blk_i[:1, E - 1:E])
    bvalid = (jax.lax.broadcasted_iota(jnp.int32, (NB, 1), 0)
              < total_used).astype(jnp.int32)
    be_ref[...] = bexp
    bv_ref[...] = bvalid


_SC_MESH = plsc.VectorSubcoreMesh(core_axis_name="c", subcore_axis_name="s")
_NW = 32
_TPW = T // _NW
_APW = A // _NW


@functools.partial(
    pl.kernel,
    out_type=jax.ShapeDtypeStruct((A_PAD, D), jnp.float32),
    mesh=_SC_MESH,
    scratch_types=[
        pltpu.VMEM((_TPW,), jnp.int32),
        pltpu.VMEM((_TPW,), jnp.int32),
        pltpu.VMEM((_TPW, D), jnp.float32),
        pltpu.SemaphoreType.DMA,
    ],
)
def _sc_scatter(x_hbm, idx_hbm, out_hbm, idx1_v, idx2_v, rows_v, sem):
    wid = lax.axis_index("s") * 2 + lax.axis_index("c")
    pltpu.sync_copy(idx_hbm.at[wid], idx1_v)
    pltpu.sync_copy(idx_hbm.at[wid + _NW], idx2_v)
    pltpu.sync_copy(x_hbm.at[0, pl.ds(wid * _TPW, _TPW)], rows_v)
    c1 = pltpu.async_copy(rows_v, out_hbm.at[idx1_v], sem)
    c2 = pltpu.async_copy(rows_v, out_hbm.at[idx2_v], sem)
    c1.wait()
    c2.wait()


@functools.partial(
    pl.kernel,
    out_type=jax.ShapeDtypeStruct((A, D), jnp.float32),
    mesh=_SC_MESH,
    scratch_types=[
        pltpu.VMEM((_APW,), jnp.int32),
        pltpu.VMEM((_APW, D), jnp.float32),
        pltpu.SemaphoreType.DMA,
    ],
)
def _sc_gather(y_hbm, idx_hbm, out_hbm, idx_v, rows_v, sem):
    wid = lax.axis_index("s") * 2 + lax.axis_index("c")
    pltpu.sync_copy(idx_hbm.at[wid], idx_v)
    pltpu.async_copy(y_hbm.at[idx_v], rows_v, sem).wait()
    pltpu.sync_copy(rows_v, out_hbm.at[pl.ds(wid * _APW, _APW)])


def _gmm_body(be_ref, bv_ref, x_ref, wg_ref, wu_ref, wd_ref, y_ref):
    i = pl.program_id(0)

    @pl.when(bv_ref[i] == 1)
    def _():
        xb = x_ref[...].astype(jnp.bfloat16)
        wg = wg_ref[0].astype(jnp.bfloat16)
        wu = wu_ref[0].astype(jnp.bfloat16)
        wd = wd_ref[0].astype(jnp.bfloat16)
        g = jnp.dot(xb, wg, preferred_element_type=jnp.float32)
        u = jnp.dot(xb, wu, preferred_element_type=jnp.float32)
        h = (g * jax.nn.sigmoid(g) * u).astype(jnp.bfloat16)
        y_ref[...] = jnp.dot(h, wd, preferred_element_type=jnp.float32)


def _combine_body(g1_ref, g2_ref, wts_ref, o_ref):
    w = wts_ref[...]
    o_ref[0] = g1_ref[...] * w[:, 0:1] + g2_ref[...] * w[:, 1:2]


def kernel(hidden_states, router_w, w_gate, w_up, w_down):
    b, s, d = hidden_states.shape

    pos, wts, bexp, bvalid = pl.pallas_call(
        _router_body,
        out_shape=[
            jax.ShapeDtypeStruct((K, T), jnp.int32),
            jax.ShapeDtypeStruct((T, K), jnp.float32),
            jax.ShapeDtypeStruct((NB, 1), jnp.int32),
            jax.ShapeDtypeStruct((NB, 1), jnp.int32),
        ],
    )(hidden_states, router_w)

    idx_scatter = pos.reshape(2 * _NW, _TPW)
    idx_gather = pos.reshape(_NW, _APW)
    bexp = bexp.reshape(NB)
    bvalid = bvalid.reshape(NB)

    x_sorted = _sc_scatter(hidden_states, idx_scatter)

    y_sorted = pl.pallas_call(
        _gmm_body,
        grid_spec=pltpu.PrefetchScalarGridSpec(
            num_scalar_prefetch=2,
            grid=(NB,),
            in_specs=[
                pl.BlockSpec((BM, D), lambda i, be, bv: (i, 0)),
                pl.BlockSpec((1, D, F), lambda i, be, bv: (be[i], 0, 0)),
                pl.BlockSpec((1, D, F), lambda i, be, bv: (be[i], 0, 0)),
                pl.BlockSpec((1, F, D), lambda i, be, bv: (be[i], 0, 0)),
            ],
            out_specs=pl.BlockSpec((BM, D), lambda i, be, bv: (i, 0)),
        ),
        out_shape=jax.ShapeDtypeStruct((A_PAD, D), jnp.float32),
    )(bexp, bvalid, x_sorted, w_gate, w_up, w_down)

    g_all = _sc_gather(y_sorted, idx_gather)

    BC = 512
    return pl.pallas_call(
        _combine_body,
        grid=(T // BC,),
        in_specs=[
            pl.BlockSpec((BC, D), lambda i: (i, 0)),
            pl.BlockSpec((BC, D), lambda i: (i + T // BC, 0)),
            pl.BlockSpec((BC, K), lambda i: (i, 0)),
        ],
        out_specs=pl.BlockSpec((1, BC, D), lambda i: (0, i, 0)),
        out_shape=jax.ShapeDtypeStruct((b, s, d), jnp.float32),
    )(g_all, g_all, wts)


# speedup vs baseline: 1.0475x; 1.0272x over previous
import functools

import jax
import jax.numpy as jnp
from jax import lax
from jax.experimental import pallas as pl
from jax.experimental.pallas import tpu as pltpu
from jax.experimental.pallas import tpu_sc as plsc

T = 2048
D = 768
F = 2048
E = 8
K = 2
BM = 512
A = T * K
A_PAD = A + E * BM
NB = A_PAD // BM


def _router_body(x_ref, rw_ref, pos_ref, wts_ref, be_ref, bv_ref):
    x = x_ref[0]
    logits = jnp.dot(x, rw_ref[...], preferred_element_type=jnp.float32)
    m = jnp.max(logits, axis=1, keepdims=True)
    ex = jnp.exp(logits - m)
    probs = ex / jnp.sum(ex, axis=1, keepdims=True)

    iota_e = jax.lax.broadcasted_iota(jnp.int32, (T, E), 1)
    m1 = jnp.max(probs, axis=1, keepdims=True)
    i1 = jnp.min(jnp.where(probs == m1, iota_e, E), axis=1, keepdims=True)
    pm = jnp.where(iota_e == i1, -1.0, probs)
    m2 = jnp.max(pm, axis=1, keepdims=True)
    i2 = jnp.min(jnp.where(pm == m2, iota_e, E), axis=1, keepdims=True)
    sw = m1 + m2
    w1 = m1 / sw
    w2 = m2 / sw
    wts_ref[...] = jnp.concatenate([w1, w2], axis=1)

    h1 = (iota_e == i1).astype(jnp.float32)
    h2 = (iota_e == i2).astype(jnp.float32)
    cc = jnp.concatenate([h1, h2], axis=1)
    sh = 1
    while sh < T:
        cc = cc + jnp.concatenate(
            [jnp.zeros((sh, 2 * E), jnp.float32), cc[:T - sh]], axis=0)
        sh *= 2
    c1 = cc[:, :E]
    c2 = cc[:, E:]
    tot1 = c1[T - 1:T, :]
    tot2 = c2[T - 1:T, :]
    counts = tot1 + tot2
    nblk = jnp.ceil(counts / BM)
    tri_ex = (jax.lax.broadcasted_iota(jnp.int32, (E, E), 0)
              < jax.lax.broadcasted_iota(jnp.int32, (E, E), 1)).astype(jnp.float32)
    tri_in = (jax.lax.broadcasted_iota(jnp.int32, (E, E), 0)
              <= jax.lax.broadcasted_iota(jnp.int32, (E, E), 1)).astype(jnp.float32)
    starts = jnp.dot(nblk * BM, tri_ex, preferred_element_type=jnp.float32)
    end_blk = jnp.dot(nblk, tri_in, preferred_element_type=jnp.float32)

    rank1 = jnp.sum(jnp.where(iota_e == i1, c1 - 1.0, 0.0), axis=1, keepdims=True)
    rank2 = jnp.sum(jnp.where(iota_e == i2, tot1 + c2 - 1.0, 0.0), axis=1,
                    keepdims=True)
    s1 = jnp.sum(jnp.where(iota_e == i1, starts, 0.0), axis=1, keepdims=True)
    s2 = jnp.sum(jnp.where(iota_e == i2, starts, 0.0), axis=1, keepdims=True)
    pos1 = (s1 + rank1).astype(jnp.int32)
    pos2 = (s2 + rank2).astype(jnp.int32)
    pos_ref[...] = jnp.concatenate(
        [pos1.reshape(1, T), pos2.reshape(1, T)], axis=0)

    end_blk_i = end_blk.astype(jnp.int32)
    bi = jax.lax.broadcasted_iota(jnp.int32, (NB, E), 0)
    ge = (bi >= end_blk_i).astype(jnp.int32)
    bexp = jnp.minimum(jnp.sum(ge, axis=1, keepdims=True), E - 1)
    total_used = jnp.sum(end_blk_i[:1, E - 1:E])
    bvalid = (jax.lax.broadcasted_iota(jnp.int32, (NB, 1), 0)
              < total_used).astype(jnp.int32)
    be_ref[...] = bexp
    bv_ref[...] = bvalid


_SC_MESH = plsc.VectorSubcoreMesh(core_axis_name="c", subcore_axis_name="s")
_NW = 32
_TPW = T // _NW
_APW = A // _NW


@functools.partial(
    pl.kernel,
    out_type=jax.ShapeDtypeStruct((A_PAD, D), jnp.float32),
    mesh=_SC_MESH,
    scratch_types=[
        pltpu.VMEM((_TPW,), jnp.int32),
        pltpu.VMEM((_TPW,), jnp.int32),
        pltpu.VMEM((_TPW, D), jnp.float32),
        pltpu.SemaphoreType.DMA,
    ],
)
def _sc_scatter(x_hbm, idx_hbm, out_hbm, idx1_v, idx2_v, rows_v, sem):
    wid = lax.axis_index("s") * 2 + lax.axis_index("c")
    pltpu.sync_copy(idx_hbm.at[wid], idx1_v)
    pltpu.sync_copy(idx_hbm.at[wid + _NW], idx2_v)
    pltpu.sync_copy(x_hbm.at[0, pl.ds(wid * _TPW, _TPW)], rows_v)
    c1 = pltpu.async_copy(rows_v, out_hbm.at[idx1_v], sem)
    c2 = pltpu.async_copy(rows_v, out_hbm.at[idx2_v], sem)
    c1.wait()
    c2.wait()


@functools.partial(
    pl.kernel,
    out_type=jax.ShapeDtypeStruct((A, D), jnp.float32),
    mesh=_SC_MESH,
    scratch_types=[
        pltpu.VMEM((_APW,), jnp.int32),
        pltpu.VMEM((_APW, D), jnp.float32),
        pltpu.SemaphoreType.DMA,
    ],
)
def _sc_gather(y_hbm, idx_hbm, out_hbm, idx_v, rows_v, sem):
    wid = lax.axis_index("s") * 2 + lax.axis_index("c")
    pltpu.sync_copy(idx_hbm.at[wid], idx_v)
    pltpu.async_copy(y_hbm.at[idx_v], rows_v, sem).wait()
    pltpu.sync_copy(rows_v, out_hbm.at[pl.ds(wid * _APW, _APW)])


def _gmm_body(be_ref, bv_ref, x_ref, wg_ref, wu_ref, wd_ref, y_ref):
    i = pl.program_id(0)

    @pl.when(bv_ref[i] == 1)
    def _():
        xb = x_ref[...].astype(jnp.bfloat16)
        wg = wg_ref[0].astype(jnp.bfloat16)
        wu = wu_ref[0].astype(jnp.bfloat16)
        wd = wd_ref[0].astype(jnp.bfloat16)
        g = jnp.dot(xb, wg, preferred_element_type=jnp.float32)
        u = jnp.dot(xb, wu, preferred_element_type=jnp.float32)
        h = (g * jax.nn.sigmoid(g) * u).astype(jnp.bfloat16)
        y_ref[...] = jnp.dot(h, wd, preferred_element_type=jnp.float32)


def _combine_body(g1_ref, g2_ref, wts_ref, o_ref):
    w = wts_ref[...]
    o_ref[0] = g1_ref[...] * w[:, 0:1] + g2_ref[...] * w[:, 1:2]


def kernel(hidden_states, router_w, w_gate, w_up, w_down):
    b, s, d = hidden_states.shape

    pos, wts, bexp, bvalid = pl.pallas_call(
        _router_body,
        out_shape=[
            jax.ShapeDtypeStruct((K, T), jnp.int32),
            jax.ShapeDtypeStruct((T, K), jnp.float32),
            jax.ShapeDtypeStruct((NB, 1), jnp.int32),
            jax.ShapeDtypeStruct((NB, 1), jnp.int32),
        ],
    )(hidden_states, router_w)

    idx_scatter = pos.reshape(2 * _NW, _TPW)
    idx_gather = pos.reshape(_NW, _APW)
    bexp = bexp.reshape(NB)
    bvalid = bvalid.reshape(NB)

    x_sorted = _sc_scatter(hidden_states, idx_scatter)

    y_sorted = pl.pallas_call(
        _gmm_body,
        grid_spec=pltpu.PrefetchScalarGridSpec(
            num_scalar_prefetch=2,
            grid=(NB,),
            in_specs=[
                pl.BlockSpec((BM, D),
                             lambda i, be, bv: (jnp.where(bv[i] == 1, i, 0), 0)),
                pl.BlockSpec((1, D, F), lambda i, be, bv: (be[i], 0, 0)),
                pl.BlockSpec((1, D, F), lambda i, be, bv: (be[i], 0, 0)),
                pl.BlockSpec((1, F, D), lambda i, be, bv: (be[i], 0, 0)),
            ],
            out_specs=pl.BlockSpec(
                (BM, D), lambda i, be, bv: (jnp.where(bv[i] == 1, i, NB), 0)),
        ),
        out_shape=jax.ShapeDtypeStruct((A_PAD + BM, D), jnp.float32),
    )(bexp, bvalid, x_sorted, w_gate, w_up, w_down)

    g_all = _sc_gather(y_sorted, idx_gather)

    BC = 512
    return pl.pallas_call(
        _combine_body,
        grid=(T // BC,),
        in_specs=[
            pl.BlockSpec((BC, D), lambda i: (i, 0)),
            pl.BlockSpec((BC, D), lambda i: (i + T // BC, 0)),
            pl.BlockSpec((BC, K), lambda i: (i, 0)),
        ],
        out_specs=pl.BlockSpec((1, BC, D), lambda i: (0, i, 0)),
        out_shape=jax.ShapeDtypeStruct((b, s, d), jnp.float32),
    )(g_all, g_all, wts)


# speedup vs baseline: 1.0648x; 1.0165x over previous
import functools

import jax
import jax.numpy as jnp
from jax import lax
from jax.experimental import pallas as pl
from jax.experimental.pallas import tpu as pltpu
from jax.experimental.pallas import tpu_sc as plsc

T = 2048
D = 768
F = 2048
E = 8
K = 2
BM = 512
A = T * K
A_PAD = A + E * BM
NB = A_PAD // BM


def _router_body(x_ref, rw_ref, pos_ref, tok_ref, wts_ref, be_ref, bv_ref,
                 br_ref):
    x = x_ref[0]
    logits = jnp.dot(x, rw_ref[...], preferred_element_type=jnp.float32)
    m = jnp.max(logits, axis=1, keepdims=True)
    ex = jnp.exp(logits - m)
    probs = ex / jnp.sum(ex, axis=1, keepdims=True)

    iota_e = jax.lax.broadcasted_iota(jnp.int32, (T, E), 1)
    m1 = jnp.max(probs, axis=1, keepdims=True)
    i1 = jnp.min(jnp.where(probs == m1, iota_e, E), axis=1, keepdims=True)
    pm = jnp.where(iota_e == i1, -1.0, probs)
    m2 = jnp.max(pm, axis=1, keepdims=True)
    i2 = jnp.min(jnp.where(pm == m2, iota_e, E), axis=1, keepdims=True)
    sw = m1 + m2
    w1 = m1 / sw
    w2 = m2 / sw

    h1 = (iota_e == i1).astype(jnp.float32)
    h2 = (iota_e == i2).astype(jnp.float32)
    cc = jnp.concatenate([h1, h2], axis=1)
    sh = 1
    while sh < T:
        cc = cc + jnp.concatenate(
            [jnp.zeros((sh, 2 * E), jnp.float32), cc[:T - sh]], axis=0)
        sh *= 2
    c1 = cc[:, :E]
    c2 = cc[:, E:]
    tot1 = c1[T - 1:T, :]
    tot2 = c2[T - 1:T, :]
    counts = tot1 + tot2
    nblk = jnp.ceil(counts / BM)
    tri_ex = (jax.lax.broadcasted_iota(jnp.int32, (E, E), 0)
              < jax.lax.broadcasted_iota(jnp.int32, (E, E), 1)).astype(jnp.float32)
    tri_in = (jax.lax.broadcasted_iota(jnp.int32, (E, E), 0)
              <= jax.lax.broadcasted_iota(jnp.int32, (E, E), 1)).astype(jnp.float32)
    starts = jnp.dot(nblk * BM, tri_ex, preferred_element_type=jnp.float32)
    end_blk = jnp.dot(nblk, tri_in, preferred_element_type=jnp.float32)

    rank1 = jnp.sum(jnp.where(iota_e == i1, c1 - 1.0, 0.0), axis=1, keepdims=True)
    rank2 = jnp.sum(jnp.where(iota_e == i2, tot1 + c2 - 1.0, 0.0), axis=1,
                    keepdims=True)
    s1 = jnp.sum(jnp.where(iota_e == i1, starts, 0.0), axis=1, keepdims=True)
    s2 = jnp.sum(jnp.where(iota_e == i2, starts, 0.0), axis=1, keepdims=True)
    pos1 = (s1 + rank1).astype(jnp.int32)
    pos2 = (s2 + rank2).astype(jnp.int32)
    pos_ref[...] = jnp.concatenate(
        [pos1.reshape(1, T), pos2.reshape(1, T)], axis=0)

    tok_ref[...] = jnp.broadcast_to(
        jax.lax.broadcasted_iota(jnp.int32, (T, 1), 0), (T, 128))
    wts_ref[...] = jnp.concatenate(
        [jnp.broadcast_to(w1, (T, 128)), jnp.broadcast_to(w2, (T, 128))],
        axis=0)

    end_blk_i = end_blk.astype(jnp.int32)
    bi = jax.lax.broadcasted_iota(jnp.int32, (NB, E), 0)
    ge = (bi >= end_blk_i).astype(jnp.int32)
    bexp = jnp.minimum(jnp.sum(ge, axis=1, keepdims=True), E - 1)
    total_used = jnp.sum(end_blk_i[:1, E - 1:E])
    bvalid = (jax.lax.broadcasted_iota(jnp.int32, (NB, 1), 0)
              < total_used).astype(jnp.int32)
    be_ref[...] = bexp
    bv_ref[...] = bvalid
    cnt_end = starts + counts
    iota_e_nb = jax.lax.broadcasted_iota(jnp.int32, (NB, E), 1)
    cend_i = jnp.sum(jnp.where(iota_e_nb == bexp, cnt_end, 0.0), axis=1,
                     keepdims=True)
    blk_base = (jax.lax.broadcasted_iota(jnp.int32, (NB, 1), 0)
                * BM).astype(jnp.float32)
    br_ref[...] = jnp.clip(cend_i - blk_base, 0.0, BM).astype(jnp.int32)


_SC_MESH = plsc.VectorSubcoreMesh(core_axis_name="c", subcore_axis_name="s")
_NW = 32
_TPW = T // _NW
_APW = A // _NW


@functools.partial(
    pl.kernel,
    out_type=[
        jax.ShapeDtypeStruct((A_PAD, D), jnp.float32),
        jax.ShapeDtypeStruct((A_PAD, 128), jnp.int32),
        jax.ShapeDtypeStruct((A_PAD, 128), jnp.float32),
    ],
    mesh=_SC_MESH,
    scratch_types=[
        pltpu.VMEM((_TPW,), jnp.int32),
        pltpu.VMEM((_TPW,), jnp.int32),
        pltpu.VMEM((_TPW, D), jnp.float32),
        pltpu.VMEM((_TPW, 128), jnp.int32),
        pltpu.VMEM((_TPW, 128), jnp.float32),
        pltpu.VMEM((_TPW, 128), jnp.float32),
        pltpu.SemaphoreType.DMA,
    ],
)
def _sc_scatter(x_hbm, idx_hbm, tok_hbm, wts_hbm, out_hbm, tout_hbm, wout_hbm,
                idx1_v, idx2_v, rows_v, tok_v, w1_v, w2_v, sem):
    wid = lax.axis_index("s") * 2 + lax.axis_index("c")
    pltpu.sync_copy(idx_hbm.at[wid], idx1_v)
    pltpu.sync_copy(idx_hbm.at[wid + _NW], idx2_v)
    pltpu.sync_copy(x_hbm.at[0, pl.ds(wid * _TPW, _TPW)], rows_v)
    pltpu.sync_copy(tok_hbm.at[pl.ds(wid * _TPW, _TPW)], tok_v)
    pltpu.sync_copy(wts_hbm.at[pl.ds(wid * _TPW, _TPW)], w1_v)
    pltpu.sync_copy(wts_hbm.at[pl.ds(T + wid * _TPW, _TPW)], w2_v)
    copies = [
        pltpu.async_copy(rows_v, out_hbm.at[idx1_v], sem),
        pltpu.async_copy(rows_v, out_hbm.at[idx2_v], sem),
        pltpu.async_copy(tok_v, tout_hbm.at[idx1_v], sem),
        pltpu.async_copy(tok_v, tout_hbm.at[idx2_v], sem),
        pltpu.async_copy(w1_v, wout_hbm.at[idx1_v], sem),
        pltpu.async_copy(w2_v, wout_hbm.at[idx2_v], sem),
    ]
    for c in copies:
        c.wait()


def _gmm_body(be_ref, bv_ref, br_ref, x_ref, tok_ref, w_ref, wg_ref, wu_ref,
              wd_ref, o_ref):
    i = pl.program_id(0)

    @pl.when(i == 0)
    def _():
        o_ref[0] = jnp.zeros((T, D), jnp.float32)

    @pl.when(bv_ref[i] == 1)
    def _():
        xb = x_ref[...].astype(jnp.bfloat16)
        wg = wg_ref[0].astype(jnp.bfloat16)
        wu = wu_ref[0].astype(jnp.bfloat16)
        wd = wd_ref[0].astype(jnp.bfloat16)
        g = jnp.dot(xb, wg, preferred_element_type=jnp.float32)
        u = jnp.dot(xb, wu, preferred_element_type=jnp.float32)
        h = (g * jax.nn.sigmoid(g) * u).astype(jnp.bfloat16)
        nrows = br_ref[i]
        rmask = jax.lax.broadcasted_iota(jnp.int32, (BM, 1), 0) < nrows
        y = jnp.where(rmask, jnp.dot(h, wd, preferred_element_type=jnp.float32),
                      0.0).astype(jnp.bfloat16)
        tokrow = tok_ref[...][:, 0:1].reshape(1, BM)
        wrow = w_ref[...][:, 0:1].reshape(1, BM)
        cmask = jax.lax.broadcasted_iota(jnp.int32, (1, BM), 1) < nrows
        iota_t = jax.lax.broadcasted_iota(jnp.int32, (T, BM), 0)
        p = jnp.where((iota_t == tokrow) & cmask, wrow, 0.0).astype(jnp.bfloat16)
        o_ref[0] += jnp.dot(p, y, preferred_element_type=jnp.float32)


def kernel(hidden_states, router_w, w_gate, w_up, w_down):
    b, s, d = hidden_states.shape

    pos, tokbc, wbc, bexp, bvalid, brows = pl.pallas_call(
        _router_body,
        out_shape=[
            jax.ShapeDtypeStruct((K, T), jnp.int32),
            jax.ShapeDtypeStruct((T, 128), jnp.int32),
            jax.ShapeDtypeStruct((K * T, 128), jnp.float32),
            jax.ShapeDtypeStruct((NB, 1), jnp.int32),
            jax.ShapeDtypeStruct((NB, 1), jnp.int32),
            jax.ShapeDtypeStruct((NB, 1), jnp.int32),
        ],
    )(hidden_states, router_w)

    idx_scatter = pos.reshape(2 * _NW, _TPW)
    bexp = bexp.reshape(NB)
    bvalid = bvalid.reshape(NB)
    brows = brows.reshape(NB)

    x_sorted, tok_sorted, w_sorted = _sc_scatter(
        hidden_states, idx_scatter, tokbc, wbc)

    out = pl.pallas_call(
        _gmm_body,
        grid_spec=pltpu.PrefetchScalarGridSpec(
            num_scalar_prefetch=3,
            grid=(NB,),
            in_specs=[
                pl.BlockSpec(
                    (BM, D),
                    lambda i, be, bv, br: (jnp.where(bv[i] == 1, i, 0), 0)),
                pl.BlockSpec(
                    (BM, 128),
                    lambda i, be, bv, br: (jnp.where(bv[i] == 1, i, 0), 0)),
                pl.BlockSpec(
                    (BM, 128),
                    lambda i, be, bv, br: (jnp.where(bv[i] == 1, i, 0), 0)),
                pl.BlockSpec((1, D, F), lambda i, be, bv, br: (be[i], 0, 0)),
                pl.BlockSpec((1, D, F), lambda i, be, bv, br: (be[i], 0, 0)),
                pl.BlockSpec((1, F, D), lambda i, be, bv, br: (be[i], 0, 0)),
            ],
            out_specs=pl.BlockSpec((1, T, D), lambda i, be, bv, br: (0, 0, 0)),
        ),
        out_shape=jax.ShapeDtypeStruct((b, s, d), jnp.float32),
    )(bexp, bvalid, brows, x_sorted, tok_sorted, w_sorted,
      w_gate, w_up, w_down)

    return out


# speedup vs baseline: 1.0659x; 1.0010x over previous
import functools

import jax
import jax.numpy as jnp
from jax import lax
from jax.experimental import pallas as pl
from jax.experimental.pallas import tpu as pltpu
from jax.experimental.pallas import tpu_sc as plsc

T = 2048
D = 768
F = 2048
E = 8
K = 2
BM = 512
A = T * K
A_PAD = A + E * BM
NB = A_PAD // BM


def _router_body(x_ref, rw_ref, pos_ref, tok_ref, wts_ref, be_ref, bv_ref,
                 br_ref):
    x = x_ref[0]
    logits = jnp.dot(x, rw_ref[...], preferred_element_type=jnp.float32)
    m = jnp.max(logits, axis=1, keepdims=True)
    ex = jnp.exp(logits - m)
    probs = ex / jnp.sum(ex, axis=1, keepdims=True)

    iota_e = jax.lax.broadcasted_iota(jnp.int32, (T, E), 1)
    m1 = jnp.max(probs, axis=1, keepdims=True)
    i1 = jnp.min(jnp.where(probs == m1, iota_e, E), axis=1, keepdims=True)
    pm = jnp.where(iota_e == i1, -1.0, probs)
    m2 = jnp.max(pm, axis=1, keepdims=True)
    i2 = jnp.min(jnp.where(pm == m2, iota_e, E), axis=1, keepdims=True)
    sw = m1 + m2
    w1 = m1 / sw
    w2 = m2 / sw

    h1 = (iota_e == i1).astype(jnp.float32)
    h2 = (iota_e == i2).astype(jnp.float32)
    cc = jnp.concatenate([h1, h2], axis=1)
    sh = 1
    while sh < T:
        cc = cc + jnp.concatenate(
            [jnp.zeros((sh, 2 * E), jnp.float32), cc[:T - sh]], axis=0)
        sh *= 2
    c1 = cc[:, :E]
    c2 = cc[:, E:]
    tot1 = c1[T - 1:T, :]
    tot2 = c2[T - 1:T, :]
    counts = tot1 + tot2
    nblk = jnp.ceil(counts / BM)
    tri_ex = (jax.lax.broadcasted_iota(jnp.int32, (E, E), 0)
              < jax.lax.broadcasted_iota(jnp.int32, (E, E), 1)).astype(jnp.float32)
    tri_in = (jax.lax.broadcasted_iota(jnp.int32, (E, E), 0)
              <= jax.lax.broadcasted_iota(jnp.int32, (E, E), 1)).astype(jnp.float32)
    starts = jnp.dot(nblk * BM, tri_ex, preferred_element_type=jnp.float32)
    end_blk = jnp.dot(nblk, tri_in, preferred_element_type=jnp.float32)

    rank1 = jnp.sum(jnp.where(iota_e == i1, c1 - 1.0, 0.0), axis=1, keepdims=True)
    rank2 = jnp.sum(jnp.where(iota_e == i2, tot1 + c2 - 1.0, 0.0), axis=1,
                    keepdims=True)
    s1 = jnp.sum(jnp.where(iota_e == i1, starts, 0.0), axis=1, keepdims=True)
    s2 = jnp.sum(jnp.where(iota_e == i2, starts, 0.0), axis=1, keepdims=True)
    pos1 = (s1 + rank1).astype(jnp.int32)
    pos2 = (s2 + rank2).astype(jnp.int32)
    pos_ref[...] = jnp.concatenate(
        [pos1.reshape(1, T), pos2.reshape(1, T)], axis=0)

    tok_ref[...] = jnp.broadcast_to(
        jax.lax.broadcasted_iota(jnp.int32, (T, 1), 0), (T, 128))
    wts_ref[...] = jnp.concatenate(
        [jnp.broadcast_to(w1, (T, 128)), jnp.broadcast_to(w2, (T, 128))],
        axis=0)

    end_blk_i = end_blk.astype(jnp.int32)
    bi = jax.lax.broadcasted_iota(jnp.int32, (NB, E), 0)
    ge = (bi >= end_blk_i).astype(jnp.int32)
    bexp = jnp.minimum(jnp.sum(ge, axis=1, keepdims=True), E - 1)
    total_used = jnp.sum(end_blk_i[:1, E - 1:E])
    bvalid = (jax.lax.broadcasted_iota(jnp.int32, (NB, 1), 0)
              < total_used).astype(jnp.int32)
    be_ref[...] = bexp
    bv_ref[...] = bvalid
    cnt_end = starts + counts
    iota_e_nb = jax.lax.broadcasted_iota(jnp.int32, (NB, E), 1)
    cend_i = jnp.sum(jnp.where(iota_e_nb == bexp, cnt_end, 0.0), axis=1,
                     keepdims=True)
    blk_base = (jax.lax.broadcasted_iota(jnp.int32, (NB, 1), 0)
                * BM).astype(jnp.float32)
    br_ref[...] = jnp.clip(cend_i - blk_base, 0.0, BM).astype(jnp.int32)


_SC_MESH = plsc.VectorSubcoreMesh(core_axis_name="c", subcore_axis_name="s")
_NW = 32
_TPW = T // _NW


@functools.partial(
    pl.kernel,
    out_type=[
        jax.ShapeDtypeStruct((A_PAD, D), jnp.float32),
        jax.ShapeDtypeStruct((A_PAD, 128), jnp.int32),
        jax.ShapeDtypeStruct((A_PAD, 128), jnp.float32),
    ],
    mesh=_SC_MESH,
    scratch_types=[
        pltpu.VMEM((_TPW,), jnp.int32),
        pltpu.VMEM((_TPW,), jnp.int32),
        pltpu.VMEM((_TPW, D), jnp.float32),
        pltpu.VMEM((_TPW, 128), jnp.int32),
        pltpu.VMEM((_TPW, 128), jnp.float32),
        pltpu.VMEM((_TPW, 128), jnp.float32),
        pltpu.SemaphoreType.DMA,
    ],
)
def _sc_scatter(x_hbm, idx_hbm, tok_hbm, wts_hbm, out_hbm, tout_hbm, wout_hbm,
                idx1_v, idx2_v, rows_v, tok_v, w1_v, w2_v, sem):
    wid = lax.axis_index("s") * 2 + lax.axis_index("c")
    pltpu.sync_copy(idx_hbm.at[wid], idx1_v)
    pltpu.sync_copy(idx_hbm.at[wid + _NW], idx2_v)
    pltpu.sync_copy(x_hbm.at[0, pl.ds(wid * _TPW, _TPW)], rows_v)
    pltpu.sync_copy(tok_hbm.at[pl.ds(wid * _TPW, _TPW)], tok_v)
    pltpu.sync_copy(wts_hbm.at[pl.ds(wid * _TPW, _TPW)], w1_v)
    pltpu.sync_copy(wts_hbm.at[pl.ds(T + wid * _TPW, _TPW)], w2_v)
    copies = [
        pltpu.async_copy(rows_v, out_hbm.at[idx1_v], sem),
        pltpu.async_copy(rows_v, out_hbm.at[idx2_v], sem),
        pltpu.async_copy(tok_v, tout_hbm.at[idx1_v], sem),
        pltpu.async_copy(tok_v, tout_hbm.at[idx2_v], sem),
        pltpu.async_copy(w1_v, wout_hbm.at[idx1_v], sem),
        pltpu.async_copy(w2_v, wout_hbm.at[idx2_v], sem),
    ]
    for c in copies:
        c.wait()


def _gmm_body(be_ref, bv_ref, br_ref, x_ref, tok_ref, w_ref, wg_ref, wu_ref,
              wd_ref, o_ref):
    i = pl.program_id(0)

    @pl.when(i == 0)
    def _():
        o_ref[0] = jnp.zeros((T, D), jnp.float32)

    @pl.when(bv_ref[i] == 1)
    def _():
        xb = x_ref[...].astype(jnp.bfloat16)
        wg = wg_ref[0].astype(jnp.bfloat16)
        wu = wu_ref[0].astype(jnp.bfloat16)
        wd = wd_ref[0].astype(jnp.bfloat16)
        g = jnp.dot(xb, wg, preferred_element_type=jnp.float32)
        u = jnp.dot(xb, wu, preferred_element_type=jnp.float32)
        h = (g * jax.nn.sigmoid(g) * u).astype(jnp.bfloat16)
        nrows = br_ref[i]
        rmask = jax.lax.broadcasted_iota(jnp.int32, (BM, 1), 0) < nrows
        y = jnp.where(rmask, jnp.dot(h, wd, preferred_element_type=jnp.float32),
                      0.0).astype(jnp.bfloat16)
        tokrow = tok_ref[...][:, 0:1].reshape(1, BM)
        wrow = w_ref[...][:, 0:1].reshape(1, BM)
        cmask = jax.lax.broadcasted_iota(jnp.int32, (1, BM), 1) < nrows
        iota_t = jax.lax.broadcasted_iota(jnp.int32, (T, BM), 0)
        p = jnp.where((iota_t == tokrow) & cmask, wrow, 0.0).astype(jnp.bfloat16)
        o_ref[0] += jnp.dot(p, y, preferred_element_type=jnp.float32)


def kernel(hidden_states, router_w, w_gate, w_up, w_down):
    b, s, d = hidden_states.shape

    pos, tokbc, wbc, bexp, bvalid, brows = pl.pallas_call(
        _router_body,
        out_shape=[
            jax.ShapeDtypeStruct((K, T), jnp.int32),
            jax.ShapeDtypeStruct((T, 128), jnp.int32),
            jax.ShapeDtypeStruct((K * T, 128), jnp.float32),
            jax.ShapeDtypeStruct((NB, 1), jnp.int32),
            jax.ShapeDtypeStruct((NB, 1), jnp.int32),
            jax.ShapeDtypeStruct((NB, 1), jnp.int32),
        ],
    )(hidden_states, router_w)

    idx_scatter = pos.reshape(2 * _NW, _TPW)
    bexp = bexp.reshape(NB)
    bvalid = bvalid.reshape(NB)
    brows = brows.reshape(NB)

    x_sorted, tok_sorted, w_sorted = _sc_scatter(
        hidden_states, idx_scatter, tokbc, wbc)

    out = pl.pallas_call(
        _gmm_body,
        grid_spec=pltpu.PrefetchScalarGridSpec(
            num_scalar_prefetch=3,
            grid=(NB,),
            in_specs=[
                pl.BlockSpec(
                    (BM, D),
                    lambda i, be, bv, br: (jnp.where(bv[i] == 1, i, 0), 0)),
                pl.BlockSpec(
                    (BM, 128),
                    lambda i, be, bv, br: (jnp.where(bv[i] == 1, i, 0), 0)),
                pl.BlockSpec(
                    (BM, 128),
                    lambda i, be, bv, br: (jnp.where(bv[i] == 1, i, 0), 0)),
                pl.BlockSpec((1, D, F), lambda i, be, bv, br: (be[i], 0, 0)),
                pl.BlockSpec((1, D, F), lambda i, be, bv, br: (be[i], 0, 0)),
                pl.BlockSpec((1, F, D), lambda i, be, bv, br: (be[i], 0, 0)),
            ],
            out_specs=pl.BlockSpec((1, T, D), lambda i, be, bv, br: (0, 0, 0)),
        ),
        out_shape=jax.ShapeDtypeStruct((b, s, d), jnp.float32),
    )(bexp, bvalid, brows, x_sorted, tok_sorted, w_sorted,
      w_gate, w_up, w_down)

    return out
